# baseline (speedup 1.0000x reference)
import functools

import jax
import jax.numpy as jnp
from jax import lax
from jax.experimental import pallas as pl
from jax.experimental.pallas import tpu as pltpu
from jax.experimental.pallas import tpu_sc as plsc

_N = 100000
_F = 128
_B = 1024
_NJ = 20

_NW = 32
_SC_C = 896
_SC_END = 99968
_SC_BASE = _SC_END - _NW * _SC_C
_TAIL = _N - _SC_END
_TC_CHUNK = 8912
_TC_GRID = 8

_NCHK = 224
_NCK = _SC_C // _NCHK
_UNROLL = 4


def _tc_kernel(js_ref, w_ref, et_ref, dtc_ref, p_acc):
    n = pl.program_id(0)
    p_part = lax.dot_general(
        et_ref[...].astype(jnp.bfloat16), w_ref[...].astype(jnp.bfloat16),
        (((0,), (0,)), ((), ())),
        preferred_element_type=jnp.float32)

    @pl.when(n == 0)
    def _init():
        p_acc[...] = p_part

    @pl.when(n != 0)
    def _accum():
        p_acc[...] += p_part

    @pl.when(n == _TC_GRID - 1)
    def _finish():
        js_row = js_ref[...]
        feat = lax.broadcasted_iota(jnp.int32, (_F, _NJ), 0)
        m_col = jnp.sum((js_row == feat).astype(jnp.float32), axis=1,
                        keepdims=True)
        dtc_ref[...] = jnp.dot(p_acc[...], m_col,
                               preferred_element_type=jnp.float32)


def _tc_call(js2, W, Et):
    return pl.pallas_call(
        _tc_kernel,
        grid=(_TC_GRID,),
        in_specs=[
            pl.BlockSpec((1, _NJ), lambda n: (0, 0)),
            pl.BlockSpec((_TC_CHUNK, _F), lambda n: (n, 0)),
            pl.BlockSpec((_TC_CHUNK, _F), lambda n: (n, 0)),
        ],
        out_specs=pl.BlockSpec((_F, 1), lambda n: (0, 0)),
        out_shape=jax.ShapeDtypeStruct((_F, 1), jnp.float32),
        scratch_shapes=[pltpu.VMEM((_F, _F), jnp.float32)],
        compiler_params=pltpu.CompilerParams(
            dimension_semantics=("arbitrary",),
        ),
    )(js2, W, Et)


_SC_MESH = plsc.VectorSubcoreMesh(core_axis_name="c", subcore_axis_name="s")


@functools.partial(
    pl.kernel,
    out_type=jax.ShapeDtypeStruct((_NW, _F), jnp.float32),
    mesh=_SC_MESH,
    scratch_types=[
        pltpu.VMEM((32,), jnp.int32),
        pltpu.VMEM((2, _NCHK, _F), jnp.float32),
        pltpu.VMEM((2, _NCHK, _F), jnp.float32),
        pltpu.VMEM((_F,), jnp.float32),
        pltpu.SemaphoreType.DMA,
        pltpu.SemaphoreType.DMA,
        pltpu.SemaphoreType.DMA,
        pltpu.SemaphoreType.DMA,
    ],
)
def _sc_call(js_hbm, w_hbm, et_hbm, dsc_hbm, js_v, w_buf, e_buf, d_stage,
             sw0, sw1, se0, se1):
    wid = lax.axis_index("s") * 2 + lax.axis_index("c")
    tile_base = _SC_BASE + wid * _SC_C
    lane = lax.iota(jnp.int32, 16)
    zero16 = jnp.zeros((16,), jnp.float32)
    w_sems = (sw0, sw1)
    e_sems = (se0, se1)

    pltpu.sync_copy(js_hbm, js_v.at[pl.ds(0, _NJ)])

    def _permute(v, idx):
        return v.at[idx].get(mode="promise_in_bounds")

    def _allsum(v):
        for sh in (8, 4, 2, 1):
            v = v + _permute(v, lane ^ sh)
        return v

    js0 = js_v[pl.ds(0, 16)]
    js1 = js_v[pl.ds(16, 16)]
    js_splats = []
    for k in range(_NJ):
        src = js0 if k < 16 else js1
        js_splats.append(_permute(src, jnp.full((16,), k % 16, jnp.int32)))
    m_vecs = []
    for cb in range(8):
        cvec = lane + cb * 16
        am = zero16
        for jsk in js_splats:
            am = am + jnp.where(cvec == jsk, 1.0, 0.0)
        m_vecs.append(am)

    def w_copy(ck, slot):
        base = pl.multiple_of(
            jnp.minimum(tile_base + ck * _NCHK, _SC_END - _NCHK), 8)
        return pltpu.make_async_copy(
            w_hbm.at[pl.ds(base, _NCHK), :], w_buf.at[slot], w_sems[slot])

    def e_copy(ck, slot):
        base = pl.multiple_of(
            jnp.minimum(tile_base + ck * _NCHK, _SC_END - _NCHK), 8)
        return pltpu.make_async_copy(
            et_hbm.at[pl.ds(base, _NCHK), :], e_buf.at[slot], e_sems[slot])

    def compute_chunk(slot, d_vecs):
        def nbody(ni, dvs):
            new = list(dvs)
            for u in range(_UNROLL):
                nn = ni * _UNROLL + u
                t = None
                for cb in range(8):
                    wv = w_buf[slot, nn, pl.ds(cb * 16, 16)]
                    p = wv * m_vecs[cb]
                    t = p if t is None else t + p
                s = _allsum(t)
                for rb in range(8):
                    ev = e_buf[slot, nn, pl.ds(rb * 16, 16)]
                    new[rb] = new[rb] + ev * s
            return tuple(new)
        return lax.fori_loop(0, _NCHK // _UNROLL, nbody, d_vecs)

    w_copy(0, 0).start()
    e_copy(0, 0).start()

    def pair_body(cp, d_vecs):
        ck0 = cp * 2
        w_copy(ck0 + 1, 1).start()
        e_copy(ck0 + 1, 1).start()
        w_copy(ck0, 0).wait()
        e_copy(ck0, 0).wait()
        d_vecs = compute_chunk(0, d_vecs)
        w_copy(ck0 + 2, 0).start()
        e_copy(ck0 + 2, 0).start()
        w_copy(ck0 + 1, 1).wait()
        e_copy(ck0 + 1, 1).wait()
        return compute_chunk(1, d_vecs)

    d_init = tuple(zero16 for _ in range(8))
    d_vecs = lax.fori_loop(0, _NCK // 2, pair_body, d_init)

    w_copy(0, 0).wait()
    e_copy(0, 0).wait()

    for rb in range(8):
        d_stage[pl.ds(rb * 16, 16)] = d_vecs[rb]
    pltpu.sync_copy(d_stage, dsc_hbm.at[wid])


def _fin_kernel(i_ref, js_ref, wt_ref, et_ref, dtc_ref, dsc_ref, out_ref):
    js_row = js_ref[...]
    feat = lax.broadcasted_iota(jnp.int32, (_F, _NJ), 0)
    m_col = jnp.sum((js_row == feat).astype(jnp.float32), axis=1,
                    keepdims=True)
    wt_m = jnp.dot(wt_ref[...], m_col,
                   preferred_element_type=jnp.float32)
    d_tail = lax.dot_general(et_ref[...], wt_m, (((0,), (0,)), ((), ())),
                             preferred_element_type=jnp.float32)
    ones = jnp.ones((_NW, 1), jnp.float32)
    dsc_col = lax.dot_general(dsc_ref[...], ones, (((0,), (0,)), ((), ())),
                              preferred_element_type=jnp.float32)
    d = dtc_ref[...] + dsc_col + d_tail
    lane = lax.broadcasted_iota(jnp.int32, (_B, _F), 1)
    onehot = (i_ref[...] == lane).astype(jnp.float32)
    out_ref[...] = jnp.dot(onehot, d, preferred_element_type=jnp.float32)


def _fin_call(i2, js2, W, Et, dtc, dsc):
    tail_blk = _SC_END // _TAIL
    return pl.pallas_call(
        _fin_kernel,
        grid=(1,),
        in_specs=[
            pl.BlockSpec((_B, 1), lambda n: (0, 0)),
            pl.BlockSpec((1, _NJ), lambda n: (0, 0)),
            pl.BlockSpec((_TAIL, _F), lambda n: (tail_blk, 0)),
            pl.BlockSpec((_TAIL, _F), lambda n: (tail_blk, 0)),
            pl.BlockSpec((_F, 1), lambda n: (0, 0)),
            pl.BlockSpec((_NW, _F), lambda n: (0, 0)),
        ],
        out_specs=pl.BlockSpec((_B, 1), lambda n: (0, 0)),
        out_shape=jax.ShapeDtypeStruct((_B, 1), jnp.float32),
    )(i2, js2, W, Et, dtc, dsc)


def kernel(i, js, W, E):
    i2 = i.astype(jnp.int32).reshape(_B, 1)
    js32 = js.astype(jnp.int32)
    js2 = js32.reshape(1, _NJ)
    Et = E.T
    dsc = _sc_call(js32, W, Et)
    dtc = _tc_call(js2, W, Et)
    out = _fin_call(i2, js2, W, Et, dtc, dsc)
    return out.reshape(_B)


# speedup vs baseline: 1.5005x; 1.5005x over previous
import jax
import jax.numpy as jnp
from jax import lax
from jax.experimental import pallas as pl
from jax.experimental.pallas import tpu as pltpu

_N = 100000
_F = 128
_B = 1024
_NJ = 20
_CHUNK = 10000
_GRID = 10


def _tc_kernel(i_ref, js_ref, w_ref, et_ref, out_ref, p_acc):
    n = pl.program_id(0)
    p_part = lax.dot_general(
        et_ref[...].astype(jnp.bfloat16), w_ref[...].astype(jnp.bfloat16),
        (((0,), (0,)), ((), ())),
        preferred_element_type=jnp.float32)

    @pl.when(n == 0)
    def _init():
        p_acc[...] = p_part

    @pl.when(n != 0)
    def _accum():
        p_acc[...] += p_part

    @pl.when(n == _GRID - 1)
    def _finish():
        js_row = js_ref[...]
        feat = lax.broadcasted_iota(jnp.int32, (_F, _NJ), 0)
        m_col = jnp.sum((js_row == feat).astype(jnp.float32), axis=1,
                        keepdims=True)
        d = jnp.dot(p_acc[...], m_col, preferred_element_type=jnp.float32)
        lane = lax.broadcasted_iota(jnp.int32, (_B, _F), 1)
        onehot = (i_ref[...] == lane).astype(jnp.float32)
        out_ref[...] = jnp.dot(onehot, d, preferred_element_type=jnp.float32)


def kernel(i, js, W, E):
    i2 = i.astype(jnp.int32).reshape(_B, 1)
    js2 = js.astype(jnp.int32).reshape(1, _NJ)
    Et = E.T
    out = pl.pallas_call(
        _tc_kernel,
        grid=(_GRID,),
        in_specs=[
            pl.BlockSpec((_B, 1), lambda n: (0, 0)),
            pl.BlockSpec((1, _NJ), lambda n: (0, 0)),
            pl.BlockSpec((_CHUNK, _F), lambda n: (n, 0)),
            pl.BlockSpec((_CHUNK, _F), lambda n: (n, 0)),
        ],
        out_specs=pl.BlockSpec((_B, 1), lambda n: (0, 0)),
        out_shape=jax.ShapeDtypeStruct((_B, 1), jnp.float32),
        scratch_shapes=[pltpu.VMEM((_F, _F), jnp.float32)],
        compiler_params=pltpu.CompilerParams(
            dimension_semantics=("arbitrary",),
        ),
    )(i2, js2, W, Et)
    return out.reshape(_B)
